# Initial kernel scaffold; baseline (speedup 1.0000x reference)
import jax
import jax.numpy as jnp
from jax.experimental import pallas as pl


def kernel(x):
    raise NotImplementedError("write your pallas kernel here")


# speedup vs baseline: 1.4469x; 1.4469x over previous
import jax
import jax.numpy as jnp
from jax import lax
from jax.experimental import pallas as pl


def _argmax_body(x_ref, o_ref):
    xb = x_ref[0]
    m = jnp.max(xb, axis=-1, keepdims=True)
    iota = lax.broadcasted_iota(jnp.int32, xb.shape, 1)
    big = jnp.int32(jnp.iinfo(jnp.int32).max)
    idx = jnp.min(jnp.where(xb == m, iota, big), axis=-1)
    o_ref[0] = idx[None, :]


def kernel(x):
    B, H, N = x.shape
    out = pl.pallas_call(
        _argmax_body,
        grid=(B,),
        in_specs=[pl.BlockSpec((1, H, N), lambda i: (i, 0, 0))],
        out_specs=pl.BlockSpec((1, 1, H), lambda i: (i, 0, 0)),
        out_shape=jax.ShapeDtypeStruct((B, 1, H), jnp.int32),
    )(x)
    return out.reshape(B, H).astype(jnp.int64)
